# baseline (speedup 1.0000x reference)
import functools

import jax
import jax.numpy as jnp
from jax import lax
from jax.experimental import pallas as pl
from jax.experimental.pallas import tpu as pltpu
from jax.experimental.pallas import tpu_sc as plsc

K = 64
D = 64
BETA = 0.25
BLOCK = 4096

_NC = 2
_NS = 16
_NW = _NC * _NS
_GROUP = 256


def _vq_block_kernel(x_ref, e_ref, oht_ref, idx_ref, loss_ref):
    x = x_ref[...]
    e = e_ref[...]
    b = x.shape[0]

    w2 = jnp.sum(e * e, axis=0, keepdims=True)
    x2 = jnp.sum(x * x, axis=1, keepdims=True)
    s = jax.lax.dot_general(x, e, (((1,), (1,)), ((), ())),
                            preferred_element_type=jnp.float32)
    scores = (x2 + w2) - 2.0 * s

    m = jnp.min(scores, axis=1, keepdims=True)
    eq = (scores == m).astype(jnp.float32)

    kk = jax.lax.broadcasted_iota(jnp.int32, (K, K), 0)
    ll = jax.lax.broadcasted_iota(jnp.int32, (K, K), 1)
    lower = (kk < ll).astype(jnp.float32)
    prior = jax.lax.dot_general(eq, lower, (((1,), (0,)), ((), ())),
                                preferred_element_type=jnp.float32)
    one_hot = eq * (prior == 0.0).astype(jnp.float32)

    ident = (kk == ll).astype(jnp.float32)
    oht = jax.lax.dot_general(ident, one_hot, (((1,), (1,)), ((), ())),
                              preferred_element_type=jnp.float32)
    oht_ref[...] = oht

    colt = jax.lax.broadcasted_iota(jnp.int32, (1, K), 1).astype(jnp.float32)
    idxt = jax.lax.dot_general(colt, oht, (((1,), (0,)), ((), ())),
                               preferred_element_type=jnp.float32)
    idx_ref[...] = idxt.astype(jnp.int32).reshape(b)

    e2 = jnp.sum(e * e, axis=1, keepdims=True)
    ones_row = jnp.ones((1, b), jnp.float32)
    sum_m = jax.lax.dot_general(ones_row, m, (((1,), (0,)), ((), ())),
                                preferred_element_type=jnp.float32)
    colsum = jax.lax.dot_general(ones_row, one_hot, (((1,), (0,)), ((), ())),
                                 preferred_element_type=jnp.float32)
    rhs2 = e2 - w2.reshape(K, 1)
    sum_c = jax.lax.dot_general(colsum, rhs2, (((1,), (0,)), ((), ())),
                                preferred_element_type=jnp.float32)
    loss_ref[...] = (sum_m + sum_c).reshape(1, 1, 1)


def _sc_gather_kernel(e_hbm, idx_hbm, out_hbm, e_v, idx_v, buf0, buf1, wsem):
    rows_w = idx_hbm.shape[0] // _NW
    ngroups = rows_w // _GROUP
    d2 = out_hbm.shape[2]
    rows_d1 = _GROUP // d2
    c = lax.axis_index("c")
    s = lax.axis_index("s")
    wid = s * _NC + c
    base = wid * rows_w
    pltpu.sync_copy(e_hbm, e_v)
    pltpu.sync_copy(idx_hbm.at[pl.ds(base, rows_w)], idx_v)
    lane = jax.lax.broadcasted_iota(jnp.int32, (16,), 0)
    bufs = [buf0, buf1]
    wd = [None, None]

    for g in range(ngroups):
        buf = bufs[g % 2]
        if wd[g % 2] is not None:
            wd[g % 2].wait()
        g0 = g * _GROUP

        def body(r, g0=g0, buf=buf):
            ridx = plsc.load_gather(idx_v, [jnp.full((16,), g0 + r, jnp.int32)])
            a = r // d2
            bb = r % d2
            for cc in range(D // 16):
                vals = plsc.load_gather(e_v, [ridx, lane + (cc * 16)])
                buf[a, bb, pl.ds(cc * 16, 16)] = vals

        plsc.parallel_loop(0, _GROUP, unroll=8)(body)
        wd[g % 2] = pltpu.async_copy(
            buf, out_hbm.at[wid, pl.ds(g * rows_d1, rows_d1)], wsem)
    for d in wd:
        if d is not None:
            d.wait()


def kernel(latents, embedding_weight):
    shape = latents.shape
    flat = latents.reshape(-1, D)
    n = flat.shape[0]
    nb = n // BLOCK

    oht, idx, loss = pl.pallas_call(
        _vq_block_kernel,
        grid=(nb,),
        in_specs=[
            pl.BlockSpec((BLOCK, D), lambda i: (i, 0)),
            pl.BlockSpec((K, D), lambda i: (0, 0)),
        ],
        out_specs=[
            pl.BlockSpec((K, BLOCK), lambda i: (0, i)),
            pl.BlockSpec((BLOCK,), lambda i: (i,)),
            pl.BlockSpec((1, 1, 1), lambda i: (i, 0, 0)),
        ],
        out_shape=[
            jax.ShapeDtypeStruct((K, n), jnp.float32),
            jax.ShapeDtypeStruct((n,), jnp.int32),
            jax.ShapeDtypeStruct((nb, 1, 1), jnp.float32),
        ],
        compiler_params=pltpu.CompilerParams(
            dimension_semantics=("parallel",),
        ),
    )(flat, embedding_weight)

    rows_d1 = _GROUP // shape[2]
    gather = functools.partial(
        pl.kernel,
        out_type=jax.ShapeDtypeStruct(shape, jnp.float32),
        mesh=plsc.VectorSubcoreMesh(core_axis_name="c", subcore_axis_name="s",
                                    num_cores=_NC, num_subcores=_NS),
        scratch_types=[
            pltpu.VMEM((K, D), jnp.float32),
            pltpu.VMEM((n // _NW,), jnp.int32),
            pltpu.VMEM((rows_d1, shape[2], D), jnp.float32),
            pltpu.VMEM((rows_d1, shape[2], D), jnp.float32),
            pltpu.SemaphoreType.DMA,
        ],
        compiler_params=pltpu.CompilerParams(needs_layout_passes=False),
    )(_sc_gather_kernel)
    quantized = gather(embedding_weight, idx)
    one_hot = oht.T
    indices = idx.reshape(shape[0], shape[1], shape[2])[:, None, :, :]
    vq_loss = jnp.sum(loss) * ((1.0 + BETA) / (n * D))
    return (quantized, vq_loss, one_hot, indices)


# speedup vs baseline: 1.0578x; 1.0578x over previous
import functools

import jax
import jax.numpy as jnp
from jax import lax
from jax.experimental import pallas as pl
from jax.experimental.pallas import tpu as pltpu
from jax.experimental.pallas import tpu_sc as plsc

K = 64
D = 64
BETA = 0.25
BLOCK = 8192

_NC = 2
_NS = 16
_NW = _NC * _NS
_GROUP = 256


def _vq_block_kernel(x_ref, e_ref, oht_ref, idx_ref, loss_ref):
    x = x_ref[...]
    e = e_ref[...]
    b = x.shape[0]

    w2 = jnp.sum(e * e, axis=0, keepdims=True)
    x2 = jnp.sum(x * x, axis=1, keepdims=True)
    s = jax.lax.dot_general(x, e, (((1,), (1,)), ((), ())),
                            preferred_element_type=jnp.float32)
    scores = (x2 + w2) - 2.0 * s

    m = jnp.min(scores, axis=1, keepdims=True)
    eq = (scores == m).astype(jnp.float32)

    kk = jax.lax.broadcasted_iota(jnp.int32, (K, K), 0)
    ll = jax.lax.broadcasted_iota(jnp.int32, (K, K), 1)
    lower = (kk < ll).astype(jnp.float32)
    prior = jax.lax.dot_general(eq, lower, (((1,), (0,)), ((), ())),
                                preferred_element_type=jnp.float32)
    one_hot = eq * (prior == 0.0).astype(jnp.float32)

    ident = (kk == ll).astype(jnp.float32)
    oht = jax.lax.dot_general(ident, one_hot, (((1,), (1,)), ((), ())),
                              preferred_element_type=jnp.float32)
    oht_ref[...] = oht

    colt = jax.lax.broadcasted_iota(jnp.int32, (1, K), 1).astype(jnp.float32)
    idxt = jax.lax.dot_general(colt, oht, (((1,), (0,)), ((), ())),
                               preferred_element_type=jnp.float32)
    idx_ref[...] = idxt.astype(jnp.int32).reshape(b)

    e2 = jnp.sum(e * e, axis=1, keepdims=True)
    ones_row = jnp.ones((1, b), jnp.float32)
    sum_m = jax.lax.dot_general(ones_row, m, (((1,), (0,)), ((), ())),
                                preferred_element_type=jnp.float32)
    colsum = jax.lax.dot_general(ones_row, one_hot, (((1,), (0,)), ((), ())),
                                 preferred_element_type=jnp.float32)
    rhs2 = e2 - w2.reshape(K, 1)
    sum_c = jax.lax.dot_general(colsum, rhs2, (((1,), (0,)), ((), ())),
                                preferred_element_type=jnp.float32)
    loss_ref[...] = (sum_m + sum_c).reshape(1, 1, 1)


def _sc_gather_kernel(e_hbm, idx_hbm, out_hbm, e_v, idx_v, buf0, buf1, wsem):
    rows_w = idx_hbm.shape[0] // _NW
    ngroups = rows_w // _GROUP
    d2 = out_hbm.shape[2]
    rows_d1 = _GROUP // d2
    c = lax.axis_index("c")
    s = lax.axis_index("s")
    wid = s * _NC + c
    base = wid * rows_w
    pltpu.sync_copy(e_hbm, e_v)
    pltpu.sync_copy(idx_hbm.at[pl.ds(base, rows_w)], idx_v)
    lane = jax.lax.broadcasted_iota(jnp.int32, (16,), 0)
    bufs = [buf0, buf1]
    wd = [None, None]

    for g in range(ngroups):
        buf = bufs[g % 2]
        if wd[g % 2] is not None:
            wd[g % 2].wait()
        g0 = g * _GROUP

        def body(r, g0=g0, buf=buf):
            ridx = plsc.load_gather(idx_v, [jnp.full((16,), g0 + r, jnp.int32)])
            a = r // d2
            bb = r % d2
            for cc in range(D // 16):
                vals = plsc.load_gather(e_v, [ridx, lane + (cc * 16)])
                buf[a, bb, pl.ds(cc * 16, 16)] = vals

        plsc.parallel_loop(0, _GROUP, unroll=8)(body)
        wd[g % 2] = pltpu.async_copy(
            buf, out_hbm.at[wid, pl.ds(g * rows_d1, rows_d1)], wsem)
    for d in wd:
        if d is not None:
            d.wait()


def kernel(latents, embedding_weight):
    shape = latents.shape
    flat = latents.reshape(-1, D)
    n = flat.shape[0]
    nb = n // BLOCK

    oht, idx, loss = pl.pallas_call(
        _vq_block_kernel,
        grid=(nb,),
        in_specs=[
            pl.BlockSpec((BLOCK, D), lambda i: (i, 0)),
            pl.BlockSpec((K, D), lambda i: (0, 0)),
        ],
        out_specs=[
            pl.BlockSpec((K, BLOCK), lambda i: (0, i)),
            pl.BlockSpec((BLOCK,), lambda i: (i,)),
            pl.BlockSpec((1, 1, 1), lambda i: (i, 0, 0)),
        ],
        out_shape=[
            jax.ShapeDtypeStruct((K, n), jnp.float32),
            jax.ShapeDtypeStruct((n,), jnp.int32),
            jax.ShapeDtypeStruct((nb, 1, 1), jnp.float32),
        ],
        compiler_params=pltpu.CompilerParams(
            dimension_semantics=("parallel",),
        ),
    )(flat, embedding_weight)

    rows_d1 = _GROUP // shape[2]
    gather = functools.partial(
        pl.kernel,
        out_type=jax.ShapeDtypeStruct(shape, jnp.float32),
        mesh=plsc.VectorSubcoreMesh(core_axis_name="c", subcore_axis_name="s",
                                    num_cores=_NC, num_subcores=_NS),
        scratch_types=[
            pltpu.VMEM((K, D), jnp.float32),
            pltpu.VMEM((n // _NW,), jnp.int32),
            pltpu.VMEM((rows_d1, shape[2], D), jnp.float32),
            pltpu.VMEM((rows_d1, shape[2], D), jnp.float32),
            pltpu.SemaphoreType.DMA,
        ],
        compiler_params=pltpu.CompilerParams(needs_layout_passes=False),
    )(_sc_gather_kernel)
    quantized = gather(embedding_weight, idx)
    one_hot = oht.T
    indices = idx.reshape(shape[0], shape[1], shape[2])[:, None, :, :]
    vq_loss = jnp.sum(loss) * ((1.0 + BETA) / (n * D))
    return (quantized, vq_loss, one_hot, indices)
